# baseline (speedup 1.0000x reference)
import functools

import jax
import jax.numpy as jnp
import numpy as np
from jax import lax
from jax.experimental import pallas as pl
from jax.experimental.pallas import tpu as pltpu
from jax.experimental.pallas import tpu_sc as plsc

N = 10000
E = 320000
D = 128
R = 18
B = 128
NT = 2641

NC = 2
NS = 16
EC = 80
NP = 184320
CT = NP // NS
NXP = 10240
AGGP = 10240
NB = 50
BN = N // NB

_mesh = plsc.VectorSubcoreMesh(core_axis_name="c", subcore_axis_name="s")

DW = D // 2

_GDN = lax.GatherDimensionNumbers(
    offset_dims=(), collapsed_slice_dims=(0,), start_index_map=(0,))


def _bcast_lane(v, j):
    idx = jnp.full((16, 1), j, jnp.int32)
    return lax.gather(v, idx, _GDN, (1,),
                      mode=lax.GatherScatterMode.PROMISE_IN_BOUNDS)


_NKC = E // NS // EC


def _pre_body(src_hbm, dst_hbm, typ_hbm, ent_hbm, emb_hbm, counts_hbm, x_hbm,
              cz_v, dall, tall, kr0, kr1, kr2, kr3, ones_v, idx_v, rows_v,
              counts_sm, sem, ssem):
    c = lax.axis_index("c")
    s = lax.axis_index("s")
    krings = (kr0, kr1, kr2, kr3)

    @pl.when(c == 0)
    def _counts():
        ept = E // NS
        base0 = s * ept
        ld = pltpu.async_copy(dst_hbm.at[pl.ds(base0, ept)], dall, sem)
        lt = pltpu.async_copy(typ_hbm.at[pl.ds(base0, ept)], tall, sem)

        zero16 = jnp.zeros((16,), jnp.float32)

        def zb(i, _):
            cz_v[pl.ds(i * 16, 16)] = zero16
            return 0
        lax.fori_loop(0, CT // 16, zb, 0)
        pltpu.sync_copy(cz_v, counts_sm.at[pl.ds(s * CT, CT)])

        one16 = jnp.ones((16,), jnp.float32)

        def ob(i, _):
            ones_v[pl.ds(i * 16, 16)] = one16
            return 0
        lax.fori_loop(0, EC // 16, ob, 0)

        ld.wait()
        lt.wait()
        def kk(i, _):
            sl = pl.ds(i * 16, 16)
            tall[sl] = dall[sl] * R + tall[sl]
            return 0
        lax.fori_loop(0, ept // 16, kk, 0)

        plsc.subcore_barrier()

        def _fire(ch, kr):
            b = ch * EC

            def cp(i, _):
                sl = pl.ds(i * 16, 16)
                kr[sl] = tall[pl.ds(b + i * 16, 16)]
                return 0
            lax.fori_loop(0, EC // 16, cp, 0)
            pltpu.async_copy(ones_v, counts_sm.at[kr], ssem, add=True)

        def _drain():
            pltpu.make_async_copy(ones_v, counts_sm.at[kr0], ssem).wait()

        def group(q, _):
            for i in range(4):
                ch = q * 4 + i

                @pl.when(q > 0)
                def _w():
                    _drain()
                _fire(ch, krings[i])
            return 0
        lax.fori_loop(0, _NKC // 4, group, 0)
        for i in range(2):
            _drain()
            _fire(_NKC - 2 + i, krings[i])
        for _ in range(4):
            _drain()

        plsc.subcore_barrier()
        pltpu.sync_copy(counts_sm.at[pl.ds(s * CT, CT)],
                        counts_hbm.at[pl.ds(s * CT, CT)])

    @pl.when(c == 1)
    def _xgather():
        def chunk(q, _):
            r0 = (s * 5 + q) * 128
            pltpu.sync_copy(ent_hbm.at[pl.ds(r0, 128)], idx_v)
            pltpu.async_copy(emb_hbm.at[idx_v], rows_v, sem).wait()
            pltpu.sync_copy(rows_v, x_hbm.at[pl.ds(r0, 128)])
            return 0
        lax.fori_loop(0, 5, chunk, 0)


_pre = functools.partial(
    pl.kernel,
    out_type=(jax.ShapeDtypeStruct((NP,), jnp.float32),
              jax.ShapeDtypeStruct((NXP, D), jnp.float32)),
    mesh=_mesh,
    scratch_types=[
        pltpu.VMEM((CT,), jnp.float32),
        pltpu.VMEM((E // NS,), jnp.int32),
        pltpu.VMEM((E // NS,), jnp.int32),
        pltpu.VMEM((EC,), jnp.int32),
        pltpu.VMEM((EC,), jnp.int32),
        pltpu.VMEM((EC,), jnp.int32),
        pltpu.VMEM((EC,), jnp.int32),
        pltpu.VMEM((EC,), jnp.float32),
        pltpu.VMEM((128,), jnp.int32),
        pltpu.VMEM((128, D), jnp.float32),
        pltpu.VMEM_SHARED((NP,), jnp.float32),
        pltpu.SemaphoreType.DMA,
        pltpu.SemaphoreType.DMA,
    ],
)(_pre_body)


_EPT = E // (NC * NS)
_PASS = 2000
_NPASS = _EPT // _PASS
_NKP = _PASS // EC


def _layer_body(src_hbm, dst_hbm, typ_hbm, cnt_hbm, y_hbm, aggp_hbm,
                r0v, r1v, r2v, f0v, f1v, d0, d1, gall, dall, tall, wall,
                agg_sm, gsem, ssem, csem):
    c = lax.axis_index("c")
    s = lax.axis_index("s")
    wid = s * NC + c
    rows = (r0v, r1v, r2v)
    fbufs = (f0v, f1v)
    drng = (d0, d1)
    ebase = wid * _EPT

    zero16 = jnp.zeros((16,), jnp.float32)

    def zrow(i, _):
        for k in range(8):
            f0v[i, pl.ds(k * 16, 16)] = zero16
        return 0
    lax.fori_loop(0, EC, zrow, 0)
    a0 = s * (AGGP // NS)
    for q in range(8):
        pltpu.sync_copy(f0v, agg_sm.at[pl.ds(a0 + q * EC, EC)])
    plsc.subcore_barrier()

    def one_pass(p, _):
        pbase = ebase + p * _PASS
        lg = pltpu.async_copy(src_hbm.at[pl.ds(pbase, _PASS)], gall, gsem)
        ldl = pltpu.async_copy(dst_hbm.at[pl.ds(pbase, _PASS)], dall, gsem)
        ltl = pltpu.async_copy(typ_hbm.at[pl.ds(pbase, _PASS)], tall, gsem)
        lg.wait(); ldl.wait(); ltl.wait()

        def keys(i, _):
            sl = pl.ds(i * 16, 16)
            gall[sl] = tall[sl] * N + gall[sl]
            tall[sl] = dall[sl] * R + tall[sl]
            return 0
        lax.fori_loop(0, _PASS // 16, keys, 0)

        def cfire(ch):
            b = ch * EC
            pltpu.async_copy(cnt_hbm.at[tall.at[pl.ds(b, EC)]],
                             wall.at[pl.ds(b, EC)], csem)

        def cdrain():
            pltpu.make_async_copy(cnt_hbm.at[tall.at[pl.ds(0, EC)]],
                                  wall.at[pl.ds(0, EC)], csem).wait()

        def cgo(ch, _):
            cfire(ch)

            @pl.when(ch >= 8)
            def _w():
                cdrain()
            return 0
        lax.fori_loop(0, _NKP, cgo, 0)
        for _ in range(8):
            cdrain()

        def winv(i, _):
            sl = pl.ds(i * 16, 16)
            wall[sl] = 1.0 / jnp.maximum(wall[sl], 1.0)
            return 0
        lax.fori_loop(0, _PASS // 16, winv, 0)

        def gfire(ch, buf):
            pltpu.async_copy(y_hbm.at[gall.at[pl.ds(ch * EC, EC)]], buf, gsem)

        def gdrain(buf):
            pltpu.make_async_copy(y_hbm.at[gall.at[pl.ds(0, EC)]],
                                  buf, gsem).wait()

        def swait(fb, db):
            pltpu.make_async_copy(fb, agg_sm.at[db], ssem).wait()

        def step(ch, rb, fb, db, sw, gn):
            b = ch * EC
            gn()
            gdrain(rb)
            sw()

            def cp(i, _):
                sl = pl.ds(i * 16, 16)
                db[sl] = dall[pl.ds(b + i * 16, 16)]
                return 0
            lax.fori_loop(0, EC // 16, cp, 0)

            hi_mask = jnp.full((16,), -65536, jnp.int32)

            def scale(i, _):
                wv = wall[pl.ds(b + i * 16, 16)]
                for jj in range(16):
                    w16 = _bcast_lane(wv, jj)
                    j = i * 16 + jj
                    for k in range(4):
                        words = rb[j, pl.ds(k * 16, 16)]
                        lo = lax.bitcast_convert_type(
                            lax.shift_left(words, 16), jnp.float32)
                        hi = lax.bitcast_convert_type(
                            words & hi_mask, jnp.float32)
                        fb[j, pl.ds(k * 32, 16)] = lo * w16
                        fb[j, pl.ds(k * 32 + 16, 16)] = hi * w16
                return 0
            lax.fori_loop(0, EC // 16, scale, 0)
            pltpu.async_copy(fb, agg_sm.at[db], ssem, add=True)

        gfire(0, rows[0])
        gfire(1, rows[1])

        def group(q, _):
            for i in range(6):
                ch = q * 6 + i

                def _sw(i=i, q=q):
                    if i < 2:
                        @pl.when(q > 0)
                        def _w():
                            swait(fbufs[i % 2], drng[i % 2])
                    else:
                        swait(fbufs[i % 2], drng[i % 2])

                def _gn(ch=ch, i=i):
                    @pl.when(ch + 2 < _NKP)
                    def _f():
                        gfire(ch + 2, rows[(i + 2) % 3])
                step(ch, rows[i % 3], fbufs[i % 2], drng[i % 2], _sw, _gn)
            return 0
        lax.fori_loop(0, _NKP // 6, group, 0)
        step(24, rows[0], fbufs[0], drng[0],
             lambda: swait(fbufs[0], drng[0]), lambda: None)
        swait(fbufs[1], drng[1])
        swait(fbufs[0], drng[0])
        return 0
    lax.fori_loop(0, _NPASS, one_pass, 0)

    plsc.subcore_barrier()
    for q in range(5):
        pltpu.sync_copy(agg_sm.at[pl.ds(a0 + q * 128, 128)],
                        aggp_hbm.at[c, pl.ds(a0 + q * 128, 128)])


_layer = functools.partial(
    pl.kernel,
    out_type=jax.ShapeDtypeStruct((NC, AGGP, D), jnp.float32),
    mesh=_mesh,
    compiler_params=pltpu.CompilerParams(use_tc_tiling_on_sc=False),
    scratch_types=[
        pltpu.VMEM((EC, DW), jnp.int32),
        pltpu.VMEM((EC, DW), jnp.int32),
        pltpu.VMEM((EC, DW), jnp.int32),
        pltpu.VMEM((EC, D), jnp.float32),
        pltpu.VMEM((EC, D), jnp.float32),
        pltpu.VMEM((EC,), jnp.int32),
        pltpu.VMEM((EC,), jnp.int32),
        pltpu.VMEM((_PASS,), jnp.int32),
        pltpu.VMEM((_PASS,), jnp.int32),
        pltpu.VMEM((_PASS,), jnp.int32),
        pltpu.VMEM((_PASS,), jnp.float32),
        pltpu.VMEM_SHARED((AGGP, D), jnp.float32),
        pltpu.SemaphoreType.DMA,
        pltpu.SemaphoreType.DMA,
        pltpu.SemaphoreType.DMA,
    ],
)(_layer_body)


def _pack_rows(y):
    bits = lax.bitcast_convert_type(
        y.astype(jnp.bfloat16).astype(jnp.float32), jnp.int32)
    parts = []
    for g in range(4):
        lo = lax.shift_right_logical(bits[:, 32 * g:32 * g + 16], 16)
        hi = bits[:, 32 * g + 16:32 * g + 32]
        parts.append(lo | hi)
    return jnp.concatenate(parts, axis=1)


def _mm1_body(x_ref, w_ref, b_ref, y_ref, z_ref):
    xb = x_ref[...]
    for rr in range(R):
        y_ref[rr] = _pack_rows(
            jnp.dot(xb, w_ref[rr], preferred_element_type=jnp.float32))
    z_ref[...] = (jnp.dot(xb, w_ref[R], preferred_element_type=jnp.float32)
                  + b_ref[0][None, :])


def _mm3_body(a0_ref, a1_ref, z_in_ref, w_ref, b_ref, y_ref, z_ref):
    xb = a0_ref[...] + a1_ref[...] + z_in_ref[...]
    for rr in range(R):
        y_ref[rr] = _pack_rows(
            jnp.dot(xb, w_ref[rr], preferred_element_type=jnp.float32))
    z_ref[...] = (jnp.dot(xb, w_ref[R], preferred_element_type=jnp.float32)
                  + b_ref[0][None, :])


_x_spec = pl.BlockSpec((BN, D), lambda i: (i, 0))
_w_spec = pl.BlockSpec((R + 1, D, D), lambda i: (0, 0, 0))
_b_spec = pl.BlockSpec((1, D), lambda i: (0, 0))
_y_spec = pl.BlockSpec((R, BN, DW), lambda i: (0, i, 0))
_z_spec = pl.BlockSpec((BN, D), lambda i: (i, 0))
_mm_out = (jax.ShapeDtypeStruct((R, N, DW), jnp.int32),
           jax.ShapeDtypeStruct((N, D), jnp.float32))

_mm1 = pl.pallas_call(
    _mm1_body, grid=(NB,),
    in_specs=[_x_spec, _w_spec, _b_spec],
    out_specs=(_y_spec, _z_spec),
    out_shape=_mm_out,
)

_mm3 = pl.pallas_call(
    _mm3_body, grid=(NB,),
    in_specs=[_x_spec, _x_spec, _x_spec, _w_spec, _b_spec],
    out_specs=(_y_spec, _z_spec),
    out_shape=_mm_out,
)


def _out_body(a0_ref, a1_ref, z_ref, bid_ref, o_ref):
    i = pl.program_id(0)

    @pl.when(i == 0)
    def _init():
        o_ref[...] = jnp.zeros_like(o_ref)

    h = a0_ref[...] + a1_ref[...] + z_ref[...]
    bid = bid_ref[0, 0, :]
    iota = lax.broadcasted_iota(jnp.int32, (B, BN), 0)
    oh = (iota == bid[None, :]).astype(jnp.float32)
    o_ref[...] += jnp.dot(oh, h, preferred_element_type=jnp.float32)


_out = pl.pallas_call(
    _out_body, grid=(NB,),
    in_specs=[_x_spec, _x_spec, _x_spec,
              pl.BlockSpec((1, 1, BN), lambda i: (i, 0, 0))],
    out_specs=pl.BlockSpec((B, D), lambda i: (0, 0)),
    out_shape=jax.ShapeDtypeStruct((B, D), jnp.float32),
)


def kernel(edge_index, edge_type, entity_ids, batch_ids, node_embeddings,
           W1, root1, b1, W2, root2, b2):
    src = edge_index[0].astype(jnp.int32)
    dst = edge_index[1].astype(jnp.int32)
    typ = edge_type.astype(jnp.int32)
    ent = jnp.pad(entity_ids.astype(jnp.int32), (0, NXP - N))

    counts, xpad = _pre(src, dst, typ, ent, node_embeddings)
    x = xpad[:N]

    wall1 = jnp.concatenate([W1, root1[None]], axis=0)
    y1, z1 = _mm1(x, wall1, b1.reshape(1, D))
    aggp1 = _layer(src, dst, typ, counts, y1.reshape(R * N, DW))

    wall2 = jnp.concatenate([W2, root2[None]], axis=0)
    y2, z2 = _mm3(aggp1[0], aggp1[1], z1, wall2, b2.reshape(1, D))
    aggp2 = _layer(src, dst, typ, counts, y2.reshape(R * N, DW))

    bid3 = batch_ids.astype(jnp.int32).reshape(NB, 1, BN)
    return _out(aggp2[0], aggp2[1], z2, bid3)


# speedup vs baseline: 2.3426x; 2.3426x over previous
import functools

import jax
import jax.numpy as jnp
import numpy as np
from jax import lax
from jax.experimental import pallas as pl
from jax.experimental.pallas import tpu as pltpu
from jax.experimental.pallas import tpu_sc as plsc

N = 10000
E = 320000
D = 128
R = 18
B = 128
NT = 2641

NC = 2
NS = 16
EC = 80
NP = 184320
CT = NP // NS
NXP = 10240
AGGP = 10240
NB = 50
BN = N // NB

_mesh = plsc.VectorSubcoreMesh(core_axis_name="c", subcore_axis_name="s")

_GDN = lax.GatherDimensionNumbers(
    offset_dims=(), collapsed_slice_dims=(0,), start_index_map=(0,))


def _bcast_lane(v, j):
    idx = jnp.full((16, 1), j, jnp.int32)
    return lax.gather(v, idx, _GDN, (1,),
                      mode=lax.GatherScatterMode.PROMISE_IN_BOUNDS)


_NKC = E // NS // EC


def _pre_body(src_hbm, dst_hbm, typ_hbm, ent_hbm, emb_hbm, counts_hbm, x_hbm,
              cz_v, dall, tall, kr0, kr1, kr2, kr3, ones_v, idx_v, rows_v,
              counts_sm, sem, ssem):
    c = lax.axis_index("c")
    s = lax.axis_index("s")
    krings = (kr0, kr1, kr2, kr3)

    @pl.when(c == 0)
    def _counts():
        ept = E // NS
        base0 = s * ept
        ld = pltpu.async_copy(dst_hbm.at[pl.ds(base0, ept)], dall, sem)
        lt = pltpu.async_copy(typ_hbm.at[pl.ds(base0, ept)], tall, sem)

        zero16 = jnp.zeros((16,), jnp.float32)

        def zb(i, _):
            cz_v[pl.ds(i * 16, 16)] = zero16
            return 0
        lax.fori_loop(0, CT // 16, zb, 0)
        pltpu.sync_copy(cz_v, counts_sm.at[pl.ds(s * CT, CT)])

        one16 = jnp.ones((16,), jnp.float32)

        def ob(i, _):
            ones_v[pl.ds(i * 16, 16)] = one16
            return 0
        lax.fori_loop(0, EC // 16, ob, 0)

        ld.wait()
        lt.wait()
        def kk(i, _):
            sl = pl.ds(i * 16, 16)
            tall[sl] = dall[sl] * R + tall[sl]
            return 0
        lax.fori_loop(0, ept // 16, kk, 0)

        plsc.subcore_barrier()

        def _fire(ch, kr):
            b = ch * EC

            def cp(i, _):
                sl = pl.ds(i * 16, 16)
                kr[sl] = tall[pl.ds(b + i * 16, 16)]
                return 0
            lax.fori_loop(0, EC // 16, cp, 0)
            pltpu.async_copy(ones_v, counts_sm.at[kr], ssem, add=True)

        def _drain():
            pltpu.make_async_copy(ones_v, counts_sm.at[kr0], ssem).wait()

        def group(q, _):
            for i in range(4):
                ch = q * 4 + i

                @pl.when(q > 0)
                def _w():
                    _drain()
                _fire(ch, krings[i])
            return 0
        lax.fori_loop(0, _NKC // 4, group, 0)
        for i in range(2):
            _drain()
            _fire(_NKC - 2 + i, krings[i])
        for _ in range(4):
            _drain()

        plsc.subcore_barrier()
        pltpu.sync_copy(counts_sm.at[pl.ds(s * CT, CT)],
                        counts_hbm.at[pl.ds(s * CT, CT)])

    @pl.when(c == 1)
    def _xgather():
        def chunk(q, _):
            r0 = (s * 5 + q) * 128
            pltpu.sync_copy(ent_hbm.at[pl.ds(r0, 128)], idx_v)
            pltpu.async_copy(emb_hbm.at[idx_v], rows_v, sem).wait()
            pltpu.sync_copy(rows_v, x_hbm.at[pl.ds(r0, 128)])
            return 0
        lax.fori_loop(0, 5, chunk, 0)


_pre = functools.partial(
    pl.kernel,
    out_type=(jax.ShapeDtypeStruct((NP,), jnp.float32),
              jax.ShapeDtypeStruct((NXP, D), jnp.float32)),
    mesh=_mesh,
    scratch_types=[
        pltpu.VMEM((CT,), jnp.float32),
        pltpu.VMEM((E // NS,), jnp.int32),
        pltpu.VMEM((E // NS,), jnp.int32),
        pltpu.VMEM((EC,), jnp.int32),
        pltpu.VMEM((EC,), jnp.int32),
        pltpu.VMEM((EC,), jnp.int32),
        pltpu.VMEM((EC,), jnp.int32),
        pltpu.VMEM((EC,), jnp.float32),
        pltpu.VMEM((128,), jnp.int32),
        pltpu.VMEM((128, D), jnp.float32),
        pltpu.VMEM_SHARED((NP,), jnp.float32),
        pltpu.SemaphoreType.DMA,
        pltpu.SemaphoreType.DMA,
    ],
)(_pre_body)


_EPT = E // (NC * NS)
_PASS = 2000
_NPASS = _EPT // _PASS
_NKP = _PASS // EC


def _layer_body(src_hbm, dst_hbm, typ_hbm, cnt_hbm, y_hbm, aggp_hbm,
                r0v, r1v, r2v, d0, d1, d2, gall, dall, tall, wall,
                agg_sm, gsem, ssem, csem, lsem):
    c = lax.axis_index("c")
    s = lax.axis_index("s")
    wid = s * NC + c
    rows = (r0v, r1v, r2v)
    drng = (d0, d1, d2)
    ebase = wid * _EPT

    zero16 = jnp.zeros((16,), jnp.float32)

    def zrow(i, _):
        for k in range(8):
            r0v[i, pl.ds(k * 16, 16)] = zero16
        return 0
    lax.fori_loop(0, EC, zrow, 0)
    a0 = s * (AGGP // NS)
    for q in range(8):
        pltpu.sync_copy(r0v, agg_sm.at[pl.ds(a0 + q * EC, EC)])
    plsc.subcore_barrier()

    def one_pass(p, _):
        pbase = ebase + p * _PASS
        lg = pltpu.async_copy(src_hbm.at[pl.ds(pbase, _PASS)], gall, lsem)
        ldl = pltpu.async_copy(dst_hbm.at[pl.ds(pbase, _PASS)], dall, lsem)
        ltl = pltpu.async_copy(typ_hbm.at[pl.ds(pbase, _PASS)], tall, lsem)
        lg.wait(); ldl.wait(); ltl.wait()

        def keys(i, _):
            sl = pl.ds(i * 16, 16)
            gall[sl] = tall[sl] * N + gall[sl]
            tall[sl] = dall[sl] * R + tall[sl]
            return 0
        lax.fori_loop(0, _PASS // 16, keys, 0)

        def cfire(ch):
            b = ch * EC
            pltpu.async_copy(cnt_hbm.at[tall.at[pl.ds(b, EC)]],
                             wall.at[pl.ds(b, EC)], csem)

        def cdrain():
            pltpu.make_async_copy(cnt_hbm.at[tall.at[pl.ds(0, EC)]],
                                  wall.at[pl.ds(0, EC)], csem).wait()

        def gfire(ch, buf):
            pltpu.async_copy(y_hbm.at[gall.at[pl.ds(ch * EC, EC)]], buf, gsem)

        def gdrain(buf):
            pltpu.make_async_copy(y_hbm.at[gall.at[pl.ds(0, EC)]],
                                  buf, gsem).wait()

        def swait(rbuf, db):
            pltpu.make_async_copy(rbuf, agg_sm.at[db], ssem).wait()

        def step(ch, rbuf, db, swaits, gnext):
            b = ch * EC
            gdrain(rbuf)
            cdrain()

            def cp(i, _):
                sl = pl.ds(i * 16, 16)
                db[sl] = dall[pl.ds(b + i * 16, 16)]
                return 0
            lax.fori_loop(0, EC // 16, cp, 0)

            def scale(i, _):
                wv = 1.0 / jnp.maximum(wall[pl.ds(b + i * 16, 16)], 1.0)
                for jj in range(16):
                    w16 = _bcast_lane(wv, jj)
                    j = i * 16 + jj
                    for k in range(8):
                        sl = pl.ds(k * 16, 16)
                        rbuf[j, sl] = rbuf[j, sl] * w16
                return 0
            lax.fori_loop(0, EC // 16, scale, 0)

            swaits()
            pltpu.async_copy(rbuf, agg_sm.at[db], ssem, add=True)
            gnext()

        cfire(0)
        cfire(1)
        cfire(2)
        gfire(0, rows[0])
        gfire(1, rows[1])

        def group(q, _):
            for i in range(3):
                ch = q * 3 + i
                prev = (i + 2) % 3

                def _sw(i=i, prev=prev, q=q):
                    if i == 0:
                        @pl.when(q > 0)
                        def _w():
                            swait(rows[prev], drng[prev])
                    else:
                        swait(rows[prev], drng[prev])

                def _gn(ch=ch, prev=prev):
                    @pl.when(ch + 2 < _NKP)
                    def _f():
                        gfire(ch + 2, rows[prev])

                    @pl.when(ch + 3 < _NKP)
                    def _c():
                        cfire(ch + 3)
                step(ch, rows[i], drng[i], _sw, _gn)
            return 0
        lax.fori_loop(0, _NKP // 3, group, 0)
        step(24, rows[0], drng[0],
             lambda: swait(rows[2], drng[2]), lambda: None)
        swait(rows[0], drng[0])
        return 0
    lax.fori_loop(0, _NPASS, one_pass, 0)

    plsc.subcore_barrier()
    for q in range(5):
        pltpu.sync_copy(agg_sm.at[pl.ds(a0 + q * 128, 128)],
                        aggp_hbm.at[c, pl.ds(a0 + q * 128, 128)])


_layer = functools.partial(
    pl.kernel,
    out_type=jax.ShapeDtypeStruct((NC, AGGP, D), jnp.float32),
    mesh=_mesh,
    scratch_types=[
        pltpu.VMEM((EC, D), jnp.float32),
        pltpu.VMEM((EC, D), jnp.float32),
        pltpu.VMEM((EC, D), jnp.float32),
        pltpu.VMEM((EC,), jnp.int32),
        pltpu.VMEM((EC,), jnp.int32),
        pltpu.VMEM((EC,), jnp.int32),
        pltpu.VMEM((_PASS,), jnp.int32),
        pltpu.VMEM((_PASS,), jnp.int32),
        pltpu.VMEM((_PASS,), jnp.int32),
        pltpu.VMEM((_PASS,), jnp.float32),
        pltpu.VMEM_SHARED((AGGP, D), jnp.float32),
        pltpu.SemaphoreType.DMA,
        pltpu.SemaphoreType.DMA,
        pltpu.SemaphoreType.DMA,
        pltpu.SemaphoreType.DMA,
    ],
)(_layer_body)


def _mm1_body(x_ref, w_ref, b_ref, y_ref, z_ref):
    xb = x_ref[...]
    for rr in range(R):
        y_ref[rr] = jnp.dot(xb, w_ref[rr], preferred_element_type=jnp.float32)
    z_ref[...] = (jnp.dot(xb, w_ref[R], preferred_element_type=jnp.float32)
                  + b_ref[0][None, :])


def _mm3_body(a0_ref, a1_ref, z_in_ref, w_ref, b_ref, y_ref, z_ref):
    xb = a0_ref[...] + a1_ref[...] + z_in_ref[...]
    for rr in range(R):
        y_ref[rr] = jnp.dot(xb, w_ref[rr], preferred_element_type=jnp.float32)
    z_ref[...] = (jnp.dot(xb, w_ref[R], preferred_element_type=jnp.float32)
                  + b_ref[0][None, :])


_x_spec = pl.BlockSpec((BN, D), lambda i: (i, 0))
_w_spec = pl.BlockSpec((R + 1, D, D), lambda i: (0, 0, 0))
_b_spec = pl.BlockSpec((1, D), lambda i: (0, 0))
_y_spec = pl.BlockSpec((R, BN, D), lambda i: (0, i, 0))
_z_spec = pl.BlockSpec((BN, D), lambda i: (i, 0))
_mm_out = (jax.ShapeDtypeStruct((R, N, D), jnp.float32),
           jax.ShapeDtypeStruct((N, D), jnp.float32))

_mm1 = pl.pallas_call(
    _mm1_body, grid=(NB,),
    in_specs=[_x_spec, _w_spec, _b_spec],
    out_specs=(_y_spec, _z_spec),
    out_shape=_mm_out,
)

_mm3 = pl.pallas_call(
    _mm3_body, grid=(NB,),
    in_specs=[_x_spec, _x_spec, _x_spec, _w_spec, _b_spec],
    out_specs=(_y_spec, _z_spec),
    out_shape=_mm_out,
)


def _out_body(a0_ref, a1_ref, z_ref, bid_ref, o_ref):
    i = pl.program_id(0)

    @pl.when(i == 0)
    def _init():
        o_ref[...] = jnp.zeros_like(o_ref)

    h = a0_ref[...] + a1_ref[...] + z_ref[...]
    bid = bid_ref[0, 0, :]
    iota = lax.broadcasted_iota(jnp.int32, (B, BN), 0)
    oh = (iota == bid[None, :]).astype(jnp.float32)
    o_ref[...] += jnp.dot(oh, h, preferred_element_type=jnp.float32)


_out = pl.pallas_call(
    _out_body, grid=(NB,),
    in_specs=[_x_spec, _x_spec, _x_spec,
              pl.BlockSpec((1, 1, BN), lambda i: (i, 0, 0))],
    out_specs=pl.BlockSpec((B, D), lambda i: (0, 0)),
    out_shape=jax.ShapeDtypeStruct((B, D), jnp.float32),
)


def kernel(edge_index, edge_type, entity_ids, batch_ids, node_embeddings,
           W1, root1, b1, W2, root2, b2):
    src = edge_index[0].astype(jnp.int32)
    dst = edge_index[1].astype(jnp.int32)
    typ = edge_type.astype(jnp.int32)
    ent = jnp.pad(entity_ids.astype(jnp.int32), (0, NXP - N))

    counts, xpad = _pre(src, dst, typ, ent, node_embeddings)
    x = xpad[:N]

    wall1 = jnp.concatenate([W1, root1[None]], axis=0)
    y1, z1 = _mm1(x, wall1, b1.reshape(1, D))
    aggp1 = _layer(src, dst, typ, counts, y1.reshape(R * N, D))

    wall2 = jnp.concatenate([W2, root2[None]], axis=0)
    y2, z2 = _mm3(aggp1[0], aggp1[1], z1, wall2, b2.reshape(1, D))
    aggp2 = _layer(src, dst, typ, counts, y2.reshape(R * N, D))

    bid3 = batch_ids.astype(jnp.int32).reshape(NB, 1, BN)
    return _out(aggp2[0], aggp2[1], z2, bid3)


# speedup vs baseline: 2.4331x; 1.0386x over previous
import functools

import jax
import jax.numpy as jnp
import numpy as np
from jax import lax
from jax.experimental import pallas as pl
from jax.experimental.pallas import tpu as pltpu
from jax.experimental.pallas import tpu_sc as plsc

N = 10000
E = 320000
D = 128
R = 18
B = 128
NT = 2641

NC = 2
NS = 16
EC = 80
NP = 184320
CT = NP // NS
NXP = 10240
AGGP = 10240
NB = 50
BN = N // NB

_mesh = plsc.VectorSubcoreMesh(core_axis_name="c", subcore_axis_name="s")

_GDN = lax.GatherDimensionNumbers(
    offset_dims=(), collapsed_slice_dims=(0,), start_index_map=(0,))


def _bcast_lane(v, j):
    idx = jnp.full((16, 1), j, jnp.int32)
    return lax.gather(v, idx, _GDN, (1,),
                      mode=lax.GatherScatterMode.PROMISE_IN_BOUNDS)


_NKC = E // NS // EC


def _xg_body(ent_hbm, emb_hbm, x_hbm, idx_v, rows_v, sem):
    c = lax.axis_index("c")
    s = lax.axis_index("s")
    wid = s * NC + c

    def chunk(q, _):
        r0 = (wid * 5 + q) * 64
        pltpu.sync_copy(ent_hbm.at[pl.ds(r0, 64)], idx_v)
        pltpu.async_copy(emb_hbm.at[idx_v], rows_v, sem).wait()
        pltpu.sync_copy(rows_v, x_hbm.at[pl.ds(r0, 64)])
        return 0
    lax.fori_loop(0, 5, chunk, 0)


_xg = functools.partial(
    pl.kernel,
    out_type=jax.ShapeDtypeStruct((NXP, D), jnp.float32),
    mesh=_mesh,
    scratch_types=[
        pltpu.VMEM((64,), jnp.int32),
        pltpu.VMEM((64, D), jnp.float32),
        pltpu.SemaphoreType.DMA,
    ],
)(_xg_body)


def _pre_body(src_hbm, dst_hbm, typ_hbm, counts_hbm,
              cz_v, dall, tall, kr0, kr1, kr2, kr3, ones_v,
              counts_sm, sem, ssem):
    c = lax.axis_index("c")
    s = lax.axis_index("s")
    krings = (kr0, kr1, kr2, kr3)

    @pl.when(c == 0)
    def _counts():
        ept = E // NS
        base0 = s * ept
        ld = pltpu.async_copy(dst_hbm.at[pl.ds(base0, ept)], dall, sem)
        lt = pltpu.async_copy(typ_hbm.at[pl.ds(base0, ept)], tall, sem)

        zero16 = jnp.zeros((16,), jnp.float32)

        def zb(i, _):
            cz_v[pl.ds(i * 16, 16)] = zero16
            return 0
        lax.fori_loop(0, CT // 16, zb, 0)
        pltpu.sync_copy(cz_v, counts_sm.at[pl.ds(s * CT, CT)])

        one16 = jnp.ones((16,), jnp.float32)

        def ob(i, _):
            ones_v[pl.ds(i * 16, 16)] = one16
            return 0
        lax.fori_loop(0, EC // 16, ob, 0)

        ld.wait()
        lt.wait()
        def kk(i, _):
            sl = pl.ds(i * 16, 16)
            tall[sl] = dall[sl] * R + tall[sl]
            return 0
        lax.fori_loop(0, ept // 16, kk, 0)

        plsc.subcore_barrier()

        def _fire(ch, kr):
            b = ch * EC

            def cp(i, _):
                sl = pl.ds(i * 16, 16)
                kr[sl] = tall[pl.ds(b + i * 16, 16)]
                return 0
            lax.fori_loop(0, EC // 16, cp, 0)
            pltpu.async_copy(ones_v, counts_sm.at[kr], ssem, add=True)

        def _drain():
            pltpu.make_async_copy(ones_v, counts_sm.at[kr0], ssem).wait()

        def group(q, _):
            for i in range(4):
                ch = q * 4 + i

                @pl.when(q > 0)
                def _w():
                    _drain()
                _fire(ch, krings[i])
            return 0
        lax.fori_loop(0, _NKC // 4, group, 0)
        for i in range(2):
            _drain()
            _fire(_NKC - 2 + i, krings[i])
        for _ in range(4):
            _drain()

        plsc.subcore_barrier()
        pltpu.sync_copy(counts_sm.at[pl.ds(s * CT, CT)],
                        counts_hbm.at[pl.ds(s * CT, CT)])


_pre = functools.partial(
    pl.kernel,
    out_type=jax.ShapeDtypeStruct((NP,), jnp.float32),
    mesh=_mesh,
    scratch_types=[
        pltpu.VMEM((CT,), jnp.float32),
        pltpu.VMEM((E // NS,), jnp.int32),
        pltpu.VMEM((E // NS,), jnp.int32),
        pltpu.VMEM((EC,), jnp.int32),
        pltpu.VMEM((EC,), jnp.int32),
        pltpu.VMEM((EC,), jnp.int32),
        pltpu.VMEM((EC,), jnp.int32),
        pltpu.VMEM((EC,), jnp.float32),
        pltpu.VMEM_SHARED((NP,), jnp.float32),
        pltpu.SemaphoreType.DMA,
        pltpu.SemaphoreType.DMA,
    ],
)(_pre_body)


_EPT = E // (NC * NS)
_PASS = 2000
_NPASS = _EPT // _PASS
_NKP = _PASS // EC


def _layer_body(src_hbm, dst_hbm, typ_hbm, cnt_hbm, y_hbm, aggp_hbm,
                r0v, r1v, r2v, d0, d1, d2, gall, dall, tall, wall,
                agg_sm, gsem, ssem, csem, lsem):
    c = lax.axis_index("c")
    s = lax.axis_index("s")
    wid = s * NC + c
    rows = (r0v, r1v, r2v)
    drng = (d0, d1, d2)
    ebase = wid * _EPT

    zero16 = jnp.zeros((16,), jnp.float32)

    def zrow(i, _):
        for k in range(8):
            r0v[i, pl.ds(k * 16, 16)] = zero16
        return 0
    lax.fori_loop(0, EC, zrow, 0)
    a0 = s * (AGGP // NS)
    for q in range(8):
        pltpu.sync_copy(r0v, agg_sm.at[pl.ds(a0 + q * EC, EC)])
    plsc.subcore_barrier()

    def one_pass(p, _):
        pbase = ebase + p * _PASS
        lg = pltpu.async_copy(src_hbm.at[pl.ds(pbase, _PASS)], gall, lsem)
        ldl = pltpu.async_copy(dst_hbm.at[pl.ds(pbase, _PASS)], dall, lsem)
        ltl = pltpu.async_copy(typ_hbm.at[pl.ds(pbase, _PASS)], tall, lsem)
        lg.wait(); ldl.wait(); ltl.wait()

        def keys(i, _):
            sl = pl.ds(i * 16, 16)
            gall[sl] = tall[sl] * N + gall[sl]
            tall[sl] = dall[sl] * R + tall[sl]
            return 0
        lax.fori_loop(0, _PASS // 16, keys, 0)

        def cfire(ch):
            b = ch * EC
            pltpu.async_copy(cnt_hbm.at[tall.at[pl.ds(b, EC)]],
                             wall.at[pl.ds(b, EC)], csem)

        def cdrain():
            pltpu.make_async_copy(cnt_hbm.at[tall.at[pl.ds(0, EC)]],
                                  wall.at[pl.ds(0, EC)], csem).wait()

        def gfire(ch, buf):
            pltpu.async_copy(y_hbm.at[gall.at[pl.ds(ch * EC, EC)]], buf, gsem)

        def gdrain(buf):
            pltpu.make_async_copy(y_hbm.at[gall.at[pl.ds(0, EC)]],
                                  buf, gsem).wait()

        def swait(rbuf, db):
            pltpu.make_async_copy(rbuf, agg_sm.at[db], ssem).wait()

        def step(ch, rbuf, db, swaits, gnext):
            b = ch * EC
            gdrain(rbuf)
            cdrain()

            def cp(i, _):
                sl = pl.ds(i * 16, 16)
                db[sl] = dall[pl.ds(b + i * 16, 16)]
                return 0
            lax.fori_loop(0, EC // 16, cp, 0)

            def scale(i, _):
                wv = 1.0 / jnp.maximum(wall[pl.ds(b + i * 16, 16)], 1.0)
                for jj in range(16):
                    w16 = _bcast_lane(wv, jj)
                    j = i * 16 + jj
                    for k in range(8):
                        sl = pl.ds(k * 16, 16)
                        rbuf[j, sl] = rbuf[j, sl] * w16
                return 0
            lax.fori_loop(0, EC // 16, scale, 0)

            swaits()
            pltpu.async_copy(rbuf, agg_sm.at[db], ssem, add=True)
            gnext()

        cfire(0)
        cfire(1)
        cfire(2)
        gfire(0, rows[0])
        gfire(1, rows[1])

        def group(q, _):
            for i in range(3):
                ch = q * 3 + i
                prev = (i + 2) % 3

                def _sw(i=i, prev=prev, q=q):
                    if i == 0:
                        @pl.when(q > 0)
                        def _w():
                            swait(rows[prev], drng[prev])
                    else:
                        swait(rows[prev], drng[prev])

                def _gn(ch=ch, prev=prev):
                    @pl.when(ch + 2 < _NKP)
                    def _f():
                        gfire(ch + 2, rows[prev])

                    @pl.when(ch + 3 < _NKP)
                    def _c():
                        cfire(ch + 3)
                step(ch, rows[i], drng[i], _sw, _gn)
            return 0
        lax.fori_loop(0, _NKP // 3, group, 0)
        step(24, rows[0], drng[0],
             lambda: swait(rows[2], drng[2]), lambda: None)
        swait(rows[0], drng[0])
        return 0
    lax.fori_loop(0, _NPASS, one_pass, 0)

    plsc.subcore_barrier()
    for q in range(5):
        pltpu.sync_copy(agg_sm.at[pl.ds(a0 + q * 128, 128)],
                        aggp_hbm.at[c, pl.ds(a0 + q * 128, 128)])


_layer = functools.partial(
    pl.kernel,
    out_type=jax.ShapeDtypeStruct((NC, AGGP, D), jnp.float32),
    mesh=_mesh,
    scratch_types=[
        pltpu.VMEM((EC, D), jnp.float32),
        pltpu.VMEM((EC, D), jnp.float32),
        pltpu.VMEM((EC, D), jnp.float32),
        pltpu.VMEM((EC,), jnp.int32),
        pltpu.VMEM((EC,), jnp.int32),
        pltpu.VMEM((EC,), jnp.int32),
        pltpu.VMEM((_PASS,), jnp.int32),
        pltpu.VMEM((_PASS,), jnp.int32),
        pltpu.VMEM((_PASS,), jnp.int32),
        pltpu.VMEM((_PASS,), jnp.float32),
        pltpu.VMEM_SHARED((AGGP, D), jnp.float32),
        pltpu.SemaphoreType.DMA,
        pltpu.SemaphoreType.DMA,
        pltpu.SemaphoreType.DMA,
        pltpu.SemaphoreType.DMA,
    ],
)(_layer_body)


def _mm1_body(x_ref, w_ref, b_ref, y_ref, z_ref):
    xb = x_ref[...]
    for rr in range(R):
        y_ref[rr] = jnp.dot(xb, w_ref[rr], preferred_element_type=jnp.float32)
    z_ref[...] = (jnp.dot(xb, w_ref[R], preferred_element_type=jnp.float32)
                  + b_ref[0][None, :])


def _mm3_body(a0_ref, a1_ref, z_in_ref, w_ref, b_ref, y_ref, z_ref):
    xb = a0_ref[...] + a1_ref[...] + z_in_ref[...]
    for rr in range(R):
        y_ref[rr] = jnp.dot(xb, w_ref[rr], preferred_element_type=jnp.float32)
    z_ref[...] = (jnp.dot(xb, w_ref[R], preferred_element_type=jnp.float32)
                  + b_ref[0][None, :])


_x_spec = pl.BlockSpec((BN, D), lambda i: (i, 0))
_w_spec = pl.BlockSpec((R + 1, D, D), lambda i: (0, 0, 0))
_b_spec = pl.BlockSpec((1, D), lambda i: (0, 0))
_y_spec = pl.BlockSpec((R, BN, D), lambda i: (0, i, 0))
_z_spec = pl.BlockSpec((BN, D), lambda i: (i, 0))
_mm_out = (jax.ShapeDtypeStruct((R, N, D), jnp.float32),
           jax.ShapeDtypeStruct((N, D), jnp.float32))

_mm1 = pl.pallas_call(
    _mm1_body, grid=(NB,),
    in_specs=[_x_spec, _w_spec, _b_spec],
    out_specs=(_y_spec, _z_spec),
    out_shape=_mm_out,
)

_mm3 = pl.pallas_call(
    _mm3_body, grid=(NB,),
    in_specs=[_x_spec, _x_spec, _x_spec, _w_spec, _b_spec],
    out_specs=(_y_spec, _z_spec),
    out_shape=_mm_out,
)


def _out_body(a0_ref, a1_ref, z_ref, bid_ref, o_ref):
    i = pl.program_id(0)

    @pl.when(i == 0)
    def _init():
        o_ref[...] = jnp.zeros_like(o_ref)

    h = a0_ref[...] + a1_ref[...] + z_ref[...]
    bid = bid_ref[0, 0, :]
    iota = lax.broadcasted_iota(jnp.int32, (B, BN), 0)
    oh = (iota == bid[None, :]).astype(jnp.float32)
    o_ref[...] += jnp.dot(oh, h, preferred_element_type=jnp.float32)


_out = pl.pallas_call(
    _out_body, grid=(NB,),
    in_specs=[_x_spec, _x_spec, _x_spec,
              pl.BlockSpec((1, 1, BN), lambda i: (i, 0, 0))],
    out_specs=pl.BlockSpec((B, D), lambda i: (0, 0)),
    out_shape=jax.ShapeDtypeStruct((B, D), jnp.float32),
)


def kernel(edge_index, edge_type, entity_ids, batch_ids, node_embeddings,
           W1, root1, b1, W2, root2, b2):
    src = edge_index[0].astype(jnp.int32)
    dst = edge_index[1].astype(jnp.int32)
    typ = edge_type.astype(jnp.int32)
    ent = jnp.pad(entity_ids.astype(jnp.int32), (0, NXP - N))

    xpad = _xg(ent, node_embeddings)
    counts = _pre(src, dst, typ)
    x = xpad[:N]

    wall1 = jnp.concatenate([W1, root1[None]], axis=0)
    y1, z1 = _mm1(x, wall1, b1.reshape(1, D))
    aggp1 = _layer(src, dst, typ, counts, y1.reshape(R * N, D))

    wall2 = jnp.concatenate([W2, root2[None]], axis=0)
    y2, z2 = _mm3(aggp1[0], aggp1[1], z1, wall2, b2.reshape(1, D))
    aggp2 = _layer(src, dst, typ, counts, y2.reshape(R * N, D))

    bid3 = batch_ids.astype(jnp.int32).reshape(NB, 1, BN)
    return _out(aggp2[0], aggp2[1], z2, bid3)
